# baseline (speedup 1.0000x reference)
import jax
import jax.numpy as jnp
from jax.experimental import pallas as pl
from jax.experimental.pallas import tpu as pltpu

_BI = 512
_BJ = 8192


def _minblock_kernel(x_ref, y_ref, row_ref, col_ref):
    i = pl.program_id(0)
    j = pl.program_id(1)

    d2 = jax.lax.dot_general(
        x_ref[...], y_ref[...], (((1,), (1,)), ((), ())),
        preferred_element_type=jnp.float32,
        precision=jax.lax.Precision.DEFAULT,
    )

    row_part = jnp.min(d2, axis=1, keepdims=True)
    col_part = jnp.min(d2, axis=0, keepdims=True)

    @pl.when(j == 0)
    def _():
        row_ref[...] = row_part

    @pl.when(j != 0)
    def _():
        row_ref[...] = jnp.minimum(row_ref[...], row_part)

    csl = pl.ds(j * _BJ, _BJ)

    @pl.when(i == 0)
    def _():
        col_ref[:, csl] = col_part

    @pl.when(i != 0)
    def _():
        col_ref[:, csl] = jnp.minimum(col_ref[:, csl], col_part)


def _finalize_kernel(row_ref, col_ref, out_ref):
    r = jnp.sqrt(jnp.maximum(row_ref[...], 1e-12))
    c = jnp.sqrt(jnp.maximum(col_ref[...], 1e-12))
    n = row_ref.shape[0]
    m = col_ref.shape[1]
    out_ref[...] = (jnp.sum(r) / n + jnp.sum(c) / m).reshape(1, 1)


@jax.jit
def kernel(set1, set2):
    s1 = set1.reshape(-1, set1.shape[-1])
    s2 = set2.reshape(-1, set2.shape[-1])
    n = s1.shape[0]
    m = s2.shape[0]
    x2 = jnp.sum(s1 * s1, axis=1, keepdims=True)
    y2 = jnp.sum(s2 * s2, axis=1, keepdims=True)
    ones_n = jnp.ones((n, 1), jnp.float32)
    ones_m = jnp.ones((m, 1), jnp.float32)
    s1 = jnp.concatenate([-2.0 * s1, ones_n, x2], axis=1).astype(jnp.bfloat16)
    s2 = jnp.concatenate([s2, y2, ones_m], axis=1).astype(jnp.bfloat16)
    d = s1.shape[1]
    row_min, col_min = pl.pallas_call(
        _minblock_kernel,
        grid=(n // _BI, m // _BJ),
        in_specs=[
            pl.BlockSpec((_BI, d), lambda i, j: (i, 0)),
            pl.BlockSpec((_BJ, d), lambda i, j: (j, 0)),
        ],
        out_specs=[
            pl.BlockSpec((_BI, 1), lambda i, j: (i, 0)),
            pl.BlockSpec((1, m), lambda i, j: (0, 0)),
        ],
        out_shape=[
            jax.ShapeDtypeStruct((n, 1), jnp.float32),
            jax.ShapeDtypeStruct((1, m), jnp.float32),
        ],
    )(s1, s2)
    out = pl.pallas_call(
        _finalize_kernel,
        out_shape=jax.ShapeDtypeStruct((1, 1), jnp.float32),
    )(row_min, col_min)
    return out[0, 0]


# speedup vs baseline: 1.0432x; 1.0432x over previous
import jax
import jax.numpy as jnp
from jax.experimental import pallas as pl
from jax.experimental.pallas import tpu as pltpu

_BI = 1024


def _minblock_kernel(x_ref, y_ref, row_ref, col_ref):
    d2 = jax.lax.dot_general(
        x_ref[...], y_ref[...], (((1,), (1,)), ((), ())),
        preferred_element_type=jnp.float32,
        precision=jax.lax.Precision.DEFAULT,
    )
    row_ref[...] = jnp.min(d2, axis=1, keepdims=True)
    col_ref[...] = jnp.min(d2, axis=0, keepdims=True)[None]


def _finalize_kernel(row_ref, col_ref, out_ref):
    r = jnp.sqrt(jnp.maximum(row_ref[...], 1e-12))
    c = jnp.sqrt(jnp.maximum(jnp.min(col_ref[...], axis=0), 1e-12))
    n = row_ref.shape[0]
    m = col_ref.shape[2]
    out_ref[...] = (jnp.sum(r) / n + jnp.sum(c) / m).reshape(1, 1)


@jax.jit
def kernel(set1, set2):
    s1 = set1.reshape(-1, set1.shape[-1])
    s2 = set2.reshape(-1, set2.shape[-1])
    n = s1.shape[0]
    m = s2.shape[0]
    x2 = jnp.sum(s1 * s1, axis=1, keepdims=True)
    y2 = jnp.sum(s2 * s2, axis=1, keepdims=True)
    ones_n = jnp.ones((n, 1), jnp.float32)
    ones_m = jnp.ones((m, 1), jnp.float32)
    s1 = jnp.concatenate([-2.0 * s1, ones_n, x2], axis=1).astype(jnp.bfloat16)
    s2 = jnp.concatenate([s2, y2, ones_m], axis=1).astype(jnp.bfloat16)
    d = s1.shape[1]
    ni = n // _BI
    row_min, col_partial = pl.pallas_call(
        _minblock_kernel,
        grid=(ni,),
        in_specs=[
            pl.BlockSpec((_BI, d), lambda i: (i, 0)),
            pl.BlockSpec((m, d), lambda i: (0, 0)),
        ],
        out_specs=[
            pl.BlockSpec((_BI, 1), lambda i: (i, 0)),
            pl.BlockSpec((1, 1, m), lambda i: (i, 0, 0)),
        ],
        out_shape=[
            jax.ShapeDtypeStruct((n, 1), jnp.float32),
            jax.ShapeDtypeStruct((ni, 1, m), jnp.float32),
        ],
        compiler_params=pltpu.CompilerParams(
            dimension_semantics=("parallel",),
        ),
    )(s1, s2)
    out = pl.pallas_call(
        _finalize_kernel,
        out_shape=jax.ShapeDtypeStruct((1, 1), jnp.float32),
    )(row_min, col_partial)
    return out[0, 0]


# speedup vs baseline: 1.1450x; 1.0976x over previous
import jax
import jax.numpy as jnp
from jax.experimental import pallas as pl
from jax.experimental.pallas import tpu as pltpu

_BI = 1024


def _prep_kernel(s1_ref, s2_ref, x_ref, y_ref):
    x = s1_ref[...]
    y = s2_ref[...]
    x2 = jnp.sum(x * x, axis=1, keepdims=True)
    y2 = jnp.sum(y * y, axis=1, keepdims=True)
    ones = jnp.ones_like(x2)
    x_ref[...] = jnp.concatenate([-2.0 * x, ones, x2], axis=1).astype(jnp.bfloat16)
    y_ref[...] = jnp.concatenate([y, y2, ones], axis=1).astype(jnp.bfloat16)


def _minblock_kernel(x_ref, y_ref, row_ref, col_ref):
    d2 = jax.lax.dot_general(
        x_ref[...], y_ref[...], (((1,), (1,)), ((), ())),
        preferred_element_type=jnp.float32,
        precision=jax.lax.Precision.DEFAULT,
    )
    row_ref[...] = jnp.min(d2, axis=1, keepdims=True).T
    col_ref[...] = jnp.min(d2, axis=0, keepdims=True)[None]


def _finalize_kernel(row_ref, col_ref, out_ref):
    r = jnp.sqrt(jnp.maximum(row_ref[...], 1e-12))
    c = jnp.sqrt(jnp.maximum(jnp.min(col_ref[...], axis=0), 1e-12))
    n = row_ref.shape[1]
    m = col_ref.shape[2]
    out_ref[...] = (jnp.sum(r) / n + jnp.sum(c) / m).reshape(1, 1)


@jax.jit
def kernel(set1, set2):
    s1 = set1.reshape(-1, set1.shape[-1])
    s2 = set2.reshape(-1, set2.shape[-1])
    n, dim = s1.shape
    m = s2.shape[0]
    d = dim + 2
    s1a, s2a = pl.pallas_call(
        _prep_kernel,
        out_shape=[
            jax.ShapeDtypeStruct((n, d), jnp.bfloat16),
            jax.ShapeDtypeStruct((m, d), jnp.bfloat16),
        ],
    )(s1, s2)
    ni = n // _BI
    row_min, col_partial = pl.pallas_call(
        _minblock_kernel,
        grid=(ni,),
        in_specs=[
            pl.BlockSpec((_BI, d), lambda i: (i, 0)),
            pl.BlockSpec((m, d), lambda i: (0, 0)),
        ],
        out_specs=[
            pl.BlockSpec((1, _BI), lambda i: (0, i)),
            pl.BlockSpec((1, 1, m), lambda i: (i, 0, 0)),
        ],
        out_shape=[
            jax.ShapeDtypeStruct((1, n), jnp.float32),
            jax.ShapeDtypeStruct((ni, 1, m), jnp.float32),
        ],
        compiler_params=pltpu.CompilerParams(
            dimension_semantics=("parallel",),
        ),
    )(s1a, s2a)
    out = pl.pallas_call(
        _finalize_kernel,
        out_shape=jax.ShapeDtypeStruct((1, 1), jnp.float32),
    )(row_min, col_partial)
    return out[0, 0]
